# baseline (speedup 1.0000x reference)
import functools

import jax
import jax.numpy as jnp
from jax import lax
from jax.experimental import pallas as pl
from jax.experimental.pallas import tpu as pltpu
from jax.experimental.pallas import tpu_sc as plsc

NC = 2
NS = 16
NW = NC * NS
CHUNK = 128


@functools.partial(jax.jit, static_argnames=("b1", "b2", "dim"))
def _gather_sc(x, weight, b1, b2, dim):
    nch = (b1 * b2) // (NW * CHUNK)
    ich = b1 // CHUNK
    xt = x.T
    xp = (xt >> 1).reshape(NW, nch, CHUNK)
    h64 = ((xt & 1) << 6).reshape(NW, nch, CHUNK)
    w2 = weight.reshape(weight.shape[0] // 2, 2 * dim)
    mesh = plsc.VectorSubcoreMesh(
        core_axis_name="c", subcore_axis_name="s", num_cores=NC, num_subcores=NS
    )

    @functools.partial(
        pl.kernel,
        out_type=jax.ShapeDtypeStruct((b2, dim, b1), jnp.float32),
        mesh=mesh,
        scratch_types=[
            pltpu.VMEM((nch, CHUNK), jnp.int32),
            pltpu.VMEM((nch, CHUNK), jnp.int32),
            pltpu.VMEM((2, CHUNK, 2 * dim + 1), jnp.float32),
            pltpu.VMEM((2, dim, CHUNK), jnp.float32),
            pltpu.SemaphoreType.DMA((2,)),
            pltpu.SemaphoreType.DMA((2,)),
        ],
        compiler_params=pltpu.CompilerParams(
            use_tc_tiling_on_sc=True, needs_layout_passes=False
        ),
    )
    def k(xp_hbm, h_hbm, table_hbm, out_hbm, xp_v, h_v, gbuf, obuf, gsem, wsem):
        wid = lax.axis_index("s") * NC + lax.axis_index("c")
        pltpu.sync_copy(xp_hbm.at[wid], xp_v)
        pltpu.sync_copy(h_hbm.at[wid], h_v)
        base_m = wid * nch

        def gather_desc(c, b):
            return pltpu.make_async_copy(
                table_hbm.at[xp_v.at[c]],
                gbuf.at[b, :, pl.ds(0, 2 * dim)],
                gsem.at[b],
            )

        def write_desc(c, b):
            m = base_m + c
            j = m // ich
            i0 = (m % ich) * CHUNK
            return pltpu.make_async_copy(
                obuf.at[b], out_hbm.at[j, :, pl.ds(i0, CHUNK)], wsem.at[b]
            )

        gather_desc(0, 0).start()
        gather_desc(1, 1).start()

        def body(c, carry):
            b = c & 1
            gather_desc(c, b).wait()

            @pl.when(c >= 2)
            def _():
                write_desc(c - 2, b).wait()

            hrow = h_v.at[c]
            gb = gbuf.at[b]
            ob = obuf.at[b]
            for q0 in range(0, CHUNK, 16):
                hv = hrow[pl.ds(q0, 16)]
                rv = lax.iota(jnp.int32, 16) + q0
                for d in range(dim):
                    val = plsc.load_gather(gb, [rv, hv + d])
                    ob[d, pl.ds(q0, 16)] = val

            write_desc(c, b).start()

            @pl.when(c + 2 < nch)
            def _():
                gather_desc(c + 2, b).start()

            return carry

        lax.fori_loop(0, nch, body, 0)
        write_desc(nch - 2, 0).wait()
        write_desc(nch - 1, 1).wait()

    o2 = k(xp, h64, w2)
    return o2.transpose(2, 0, 1)


def kernel(x, weight):
    dim = weight.shape[1]
    b1, b2 = x.shape
    return _gather_sc(x, weight, b1, b2, dim)


# speedup vs baseline: 1.2205x; 1.2205x over previous
import functools

import jax
import jax.numpy as jnp
from jax import lax
from jax.experimental import pallas as pl
from jax.experimental.pallas import tpu as pltpu
from jax.experimental.pallas import tpu_sc as plsc

NC = 2
NS = 16
NW = NC * NS
ROWS_PER_CHUNK = 2
LOOKAHEAD = 5
NBUF = 10


@functools.partial(jax.jit, static_argnames=("b1", "b2", "dim"))
def _gather_sc(x_flat, weight, b1, b2, dim):
    chunk = ROWS_PER_CHUNK * b2
    nchunk = (b1 * b2) // (NW * chunk)
    idx3 = x_flat.reshape(NW, nchunk * ROWS_PER_CHUNK, b2)
    mesh = plsc.VectorSubcoreMesh(
        core_axis_name="c", subcore_axis_name="s", num_cores=NC, num_subcores=NS
    )

    @functools.partial(
        pl.kernel,
        out_type=jax.ShapeDtypeStruct((b1, b2, dim), jnp.float32),
        mesh=mesh,
        scratch_types=[
            pltpu.VMEM((nchunk * ROWS_PER_CHUNK, b2), jnp.int32),
            pltpu.VMEM((NBUF, ROWS_PER_CHUNK, b2, dim), jnp.float32),
            pltpu.SemaphoreType.DMA((NBUF,)),
            pltpu.SemaphoreType.DMA((NBUF,)),
        ],
        compiler_params=pltpu.CompilerParams(use_tc_tiling_on_sc=False),
    )
    def k(idx_hbm, table_hbm, out_hbm, idx_v, rows_v, gsem, wsem):
        wid = lax.axis_index("s") * NC + lax.axis_index("c")
        pltpu.sync_copy(idx_hbm.at[wid], idx_v)
        row_base = wid * (nchunk * ROWS_PER_CHUNK)

        def issue_gather(c):
            b = c % NBUF
            return [
                pltpu.async_copy(
                    table_hbm.at[idx_v.at[c * ROWS_PER_CHUNK + r]],
                    rows_v.at[b, r],
                    gsem.at[b],
                )
                for r in range(ROWS_PER_CHUNK)
            ]

        def issue_write(c):
            b = c % NBUF
            dst = out_hbm.at[pl.ds(row_base + c * ROWS_PER_CHUNK, ROWS_PER_CHUNK)]
            return pltpu.async_copy(rows_v.at[b], dst, wsem.at[b])

        gathers, writes = {}, {}
        for c in range(min(LOOKAHEAD, nchunk)):
            gathers[c] = issue_gather(c)
        for j in range(nchunk):
            f = j + LOOKAHEAD
            if f < nchunk:
                if f >= NBUF:
                    writes[f - NBUF].wait()
                gathers[f] = issue_gather(f)
            for g in gathers[j]:
                g.wait()
            writes[j] = issue_write(j)
        for j in range(max(0, nchunk - NBUF), nchunk):
            writes[j].wait()

    return k(idx3, weight)


def kernel(x, weight):
    dim = weight.shape[1]
    b1, b2 = x.shape
    return _gather_sc(x.reshape(-1), weight, b1, b2, dim)
